# baseline (speedup 1.0000x reference)
import functools

import jax
import jax.numpy as jnp
from jax import lax
from jax.experimental import pallas as pl
from jax.experimental.pallas import tpu as pltpu
from jax.experimental.pallas import tpu_sc as plsc

NC = 2
NS = 16
NW = NC * NS


def _make_gather_kernel(B, H):
    b_per_w = B // NW
    mesh = plsc.VectorSubcoreMesh(core_axis_name="c", subcore_axis_name="s")

    @functools.partial(
        pl.kernel,
        mesh=mesh,
        compiler_params=pltpu.CompilerParams(use_tc_tiling_on_sc=False),
        out_type=[
            jax.ShapeDtypeStruct((B, H), jnp.float32),
            jax.ShapeDtypeStruct((B, H), jnp.float32),
        ],
        scratch_types=[
            pltpu.VMEM((b_per_w,), jnp.int32),
            pltpu.VMEM((b_per_w,), jnp.int32),
            pltpu.VMEM((b_per_w, H), jnp.float32),
            pltpu.VMEM((b_per_w, H), jnp.float32),
            pltpu.SemaphoreType.DMA,
            pltpu.SemaphoreType.DMA,
        ],
    )
    def gather_k(users_hbm, items_hbm, uemb_hbm, iemb_hbm, u_out, v_out,
                 uidx_v, iidx_v, urows_v, irows_v, usem, isem):
        wid = lax.axis_index("s") * NC + lax.axis_index("c")
        base = wid * b_per_w
        pltpu.sync_copy(users_hbm.at[pl.ds(base, b_per_w)], uidx_v)
        pltpu.sync_copy(items_hbm.at[pl.ds(base, b_per_w)], iidx_v)
        cu = pltpu.async_copy(uemb_hbm.at[uidx_v], urows_v, usem)
        ci = pltpu.async_copy(iemb_hbm.at[iidx_v], irows_v, isem)
        cu.wait()
        pltpu.sync_copy(urows_v, u_out.at[pl.ds(base, b_per_w)])
        ci.wait()
        pltpu.sync_copy(irows_v, v_out.at[pl.ds(base, b_per_w)])

    return gather_k


def _mlp_body(u_ref, v_ref, w1u_ref, w1v_ref, b1_ref, w2_ref, b2_ref,
              w3_ref, b3_ref, w4_ref, b4_ref, out_ref):
    f32 = jnp.float32
    x = jnp.dot(u_ref[...], w1u_ref[...], preferred_element_type=f32)
    x += jnp.dot(v_ref[...], w1v_ref[...], preferred_element_type=f32)
    h = jnp.maximum(x + b1_ref[...], 0.0)
    h = jnp.maximum(
        jnp.dot(h, w2_ref[...], preferred_element_type=f32) + b2_ref[...], 0.0)
    h = jnp.maximum(
        jnp.dot(h, w3_ref[...], preferred_element_type=f32) + b3_ref[...], 0.0)
    logit = jnp.dot(h, w4_ref[...], preferred_element_type=f32) + b4_ref[...]
    out_ref[...] = jax.nn.sigmoid(logit) * 9.0 + 1.0


def _make_mlp_call(B, H, BLK):
    grid = (B // BLK,)
    full = lambda shape: pl.BlockSpec(shape, lambda i: (0, 0))
    return pl.pallas_call(
        _mlp_body,
        grid=grid,
        in_specs=[
            pl.BlockSpec((BLK, H), lambda i: (i, 0)),
            pl.BlockSpec((BLK, H), lambda i: (i, 0)),
            full((H, 64)),
            full((H, 64)),
            full((1, 64)),
            full((64, 32)),
            full((1, 32)),
            full((32, 16)),
            full((1, 16)),
            full((16, 1)),
            full((1, 1)),
        ],
        out_specs=pl.BlockSpec((BLK, 1), lambda i: (i, 0)),
        out_shape=jax.ShapeDtypeStruct((B, 1), jnp.float32),
    )


def kernel(users, items, user_emb, item_emb, W1, b1, W2, b2, W3, b3, W4, b4):
    B = users.shape[0]
    H = user_emb.shape[1]
    users = users.astype(jnp.int32)
    items = items.astype(jnp.int32)

    u, v = _make_gather_kernel(B, H)(users, items, user_emb, item_emb)

    w1u = W1[:, :H].T
    w1v = W1[:, H:].T
    out = _make_mlp_call(B, H, 2048)(
        u, v, w1u, w1v, b1.reshape(1, -1), W2.T, b2.reshape(1, -1),
        W3.T, b3.reshape(1, -1), W4.T, b4.reshape(1, 1))
    return out


# speedup vs baseline: 1.0020x; 1.0020x over previous
import functools

import jax
import jax.numpy as jnp
from jax import lax
from jax.experimental import pallas as pl
from jax.experimental.pallas import tpu as pltpu
from jax.experimental.pallas import tpu_sc as plsc

NC = 2
NS = 16
NW = NC * NS


def _make_gather_kernel(B, NP, H2):
    b_per_w = B // NW
    mesh = plsc.VectorSubcoreMesh(core_axis_name="c", subcore_axis_name="s")

    @functools.partial(
        pl.kernel,
        mesh=mesh,
        out_type=[
            jax.ShapeDtypeStruct((B, H2), jnp.float32),
            jax.ShapeDtypeStruct((B, H2), jnp.float32),
        ],
        scratch_types=[
            pltpu.VMEM((b_per_w,), jnp.int32),
            pltpu.VMEM((b_per_w,), jnp.int32),
            pltpu.VMEM((b_per_w, H2), jnp.float32),
            pltpu.SemaphoreType.DMA,
        ],
    )
    def gather_k(upair_hbm, ipair_hbm, uemb_hbm, iemb_hbm, u_out, v_out,
                 uidx_v, iidx_v, rows_v, sem):
        wid = lax.axis_index("s") * NC + lax.axis_index("c")
        base = wid * b_per_w
        pltpu.sync_copy(upair_hbm.at[pl.ds(base, b_per_w)], uidx_v)
        pltpu.sync_copy(ipair_hbm.at[pl.ds(base, b_per_w)], iidx_v)
        pltpu.async_copy(uemb_hbm.at[uidx_v], rows_v, sem).wait()
        pltpu.sync_copy(rows_v, u_out.at[pl.ds(base, b_per_w)])
        pltpu.async_copy(iemb_hbm.at[iidx_v], rows_v, sem).wait()
        pltpu.sync_copy(rows_v, v_out.at[pl.ds(base, b_per_w)])

    return gather_k


def _mlp_body(ur_ref, vr_ref, uq_ref, vq_ref, w1u_ref, w1v_ref, b1_ref,
              w2_ref, b2_ref, w3_ref, b3_ref, w4_ref, b4_ref, out_ref):
    f32 = jnp.float32
    H = 64
    u = jnp.where(uq_ref[...] == 0, ur_ref[:, :H], ur_ref[:, H:])
    v = jnp.where(vq_ref[...] == 0, vr_ref[:, :H], vr_ref[:, H:])
    x = jnp.dot(u, w1u_ref[...], preferred_element_type=f32)
    x += jnp.dot(v, w1v_ref[...], preferred_element_type=f32)
    h = jnp.maximum(x + b1_ref[...], 0.0)
    h = jnp.maximum(
        jnp.dot(h, w2_ref[...], preferred_element_type=f32) + b2_ref[...], 0.0)
    h = jnp.maximum(
        jnp.dot(h, w3_ref[...], preferred_element_type=f32) + b3_ref[...], 0.0)
    logit = jnp.dot(h, w4_ref[...], preferred_element_type=f32) + b4_ref[...]
    out_ref[...] = jax.nn.sigmoid(logit) * 9.0 + 1.0


def _make_mlp_call(B, H, BLK):
    grid = (B // BLK,)
    full = lambda shape: pl.BlockSpec(shape, lambda i: (0, 0))
    return pl.pallas_call(
        _mlp_body,
        grid=grid,
        in_specs=[
            pl.BlockSpec((BLK, 2 * H), lambda i: (i, 0)),
            pl.BlockSpec((BLK, 2 * H), lambda i: (i, 0)),
            pl.BlockSpec((BLK, 1), lambda i: (i, 0)),
            pl.BlockSpec((BLK, 1), lambda i: (i, 0)),
            full((H, 64)),
            full((H, 64)),
            full((1, 64)),
            full((64, 32)),
            full((1, 32)),
            full((32, 16)),
            full((1, 16)),
            full((16, 1)),
            full((1, 1)),
        ],
        out_specs=pl.BlockSpec((BLK, 1), lambda i: (i, 0)),
        out_shape=jax.ShapeDtypeStruct((B, 1), jnp.float32),
    )


def kernel(users, items, user_emb, item_emb, W1, b1, W2, b2, W3, b3, W4, b4):
    B = users.shape[0]
    N, H = user_emb.shape
    users = users.astype(jnp.int32)
    items = items.astype(jnp.int32)

    upair = lax.shift_right_logical(users, 1)
    ipair = lax.shift_right_logical(items, 1)
    uq = lax.bitwise_and(users, 1).reshape(B, 1)
    iq = lax.bitwise_and(items, 1).reshape(B, 1)
    uemb2 = user_emb.reshape(N // 2, 2 * H)
    iemb2 = item_emb.reshape(N // 2, 2 * H)

    ur, vr = _make_gather_kernel(B, N // 2, 2 * H)(upair, ipair, uemb2, iemb2)

    w1u = W1[:, :H].T
    w1v = W1[:, H:].T
    out = _make_mlp_call(B, H, 2048)(
        ur, vr, uq, iq, w1u, w1v, b1.reshape(1, -1), W2.T, b2.reshape(1, -1),
        W3.T, b3.reshape(1, -1), W4.T, b4.reshape(1, 1))
    return out


# speedup vs baseline: 2.0980x; 2.0937x over previous
import functools

import jax
import jax.numpy as jnp
from jax import lax
from jax.experimental import pallas as pl
from jax.experimental.pallas import tpu as pltpu
from jax.experimental.pallas import tpu_sc as plsc

NC = 2
NS = 16
NW = NC * NS
LANES = 128


def _make_gather_kernel(B, N, H):
    b_per_w = B // NW
    mesh = plsc.VectorSubcoreMesh(core_axis_name="c", subcore_axis_name="s")

    @functools.partial(
        pl.kernel,
        mesh=mesh,
        compiler_params=pltpu.CompilerParams(needs_layout_passes=False),
        out_type=[
            jax.ShapeDtypeStruct((B, 2 * H), jnp.float32),
            jax.ShapeDtypeStruct((B, 2 * H), jnp.float32),
        ],
        scratch_types=[
            pltpu.VMEM((b_per_w,), jnp.int32),
            pltpu.VMEM((b_per_w,), jnp.int32),
            pltpu.VMEM((H, LANES), jnp.float32),
            pltpu.VMEM((b_per_w, 2 * H), jnp.float32),
            pltpu.SemaphoreType.DMA,
        ],
    )
    def gather_k(su_hbm, up_hbm, si_hbm, ip_hbm, uembT_hbm, iembT_hbm,
                 uout, vout, sv_v, pos_v, blockbuf, outbuf, sem):
        wid = lax.axis_index("s") * NC + lax.axis_index("c")
        base = wid * b_per_w
        lane_iota = lax.iota(jnp.int32, 16)

        def one_table(sv_hbm, pos_hbm, tabT_hbm, out_hbm):
            pltpu.sync_copy(sv_hbm.at[pl.ds(base, b_per_w)], sv_v)
            pltpu.sync_copy(pos_hbm.at[pl.ds(base, b_per_w)], pos_v)

            def body(i, cur_col):
                idx_vec = sv_v[pl.ds(i * 16, 16)]
                col_vec = lax.shift_right_logical(idx_vec, 7)
                lane_vec16 = lax.bitwise_and(idx_vec, LANES - 1)
                cur = cur_col
                for j in range(16):
                    sel = lane_iota == j
                    col = jnp.sum(jnp.where(sel, col_vec, 0))
                    lane = jnp.sum(jnp.where(sel, lane_vec16, 0))

                    @pl.when(col != cur)
                    def _fetch():
                        off = pl.multiple_of(col * LANES, LANES)
                        pltpu.sync_copy(tabT_hbm.at[:, pl.ds(off, LANES)],
                                        blockbuf)

                    lane_vec = jnp.full((16,), lane, jnp.int32)
                    k = i * 16 + j
                    for jj in range(H // 16):
                        feats = lane_iota + (16 * jj)
                        vals = plsc.load_gather(blockbuf, [feats, lane_vec])
                        outbuf[k, pl.ds(16 * jj, 16)] = vals
                    cur = col
                return cur

            lax.fori_loop(0, b_per_w // 16, body, jnp.int32(-1))
            pltpu.async_copy(outbuf, out_hbm.at[pos_v], sem).wait()

        one_table(su_hbm, up_hbm, uembT_hbm, uout)
        one_table(si_hbm, ip_hbm, iembT_hbm, vout)

    return gather_k


def _mlp_body(u_ref, v_ref, w1u_ref, w1v_ref, b1_ref, w2_ref, b2_ref,
              w3_ref, b3_ref, w4_ref, b4_ref, out_ref):
    f32 = jnp.float32
    H = 64
    x = jnp.dot(u_ref[:, :H], w1u_ref[...], preferred_element_type=f32)
    x += jnp.dot(v_ref[:, :H], w1v_ref[...], preferred_element_type=f32)
    h = jnp.maximum(x + b1_ref[...], 0.0)
    h = jnp.maximum(
        jnp.dot(h, w2_ref[...], preferred_element_type=f32) + b2_ref[...], 0.0)
    h = jnp.maximum(
        jnp.dot(h, w3_ref[...], preferred_element_type=f32) + b3_ref[...], 0.0)
    logit = jnp.dot(h, w4_ref[...], preferred_element_type=f32) + b4_ref[...]
    out_ref[...] = jax.nn.sigmoid(logit) * 9.0 + 1.0


def _make_mlp_call(B, H, BLK):
    grid = (B // BLK,)
    full = lambda shape: pl.BlockSpec(shape, lambda i: (0, 0))
    return pl.pallas_call(
        _mlp_body,
        grid=grid,
        in_specs=[
            pl.BlockSpec((BLK, 2 * H), lambda i: (i, 0)),
            pl.BlockSpec((BLK, 2 * H), lambda i: (i, 0)),
            full((H, 64)),
            full((H, 64)),
            full((1, 64)),
            full((64, 32)),
            full((1, 32)),
            full((32, 16)),
            full((1, 16)),
            full((16, 1)),
            full((1, 1)),
        ],
        out_specs=pl.BlockSpec((BLK, 1), lambda i: (i, 0)),
        out_shape=jax.ShapeDtypeStruct((B, 1), jnp.float32),
    )


def kernel(users, items, user_emb, item_emb, W1, b1, W2, b2, W3, b3, W4, b4):
    B = users.shape[0]
    N, H = user_emb.shape
    users = users.astype(jnp.int32)
    items = items.astype(jnp.int32)

    iota = lax.iota(jnp.int32, B)
    su, up = lax.sort_key_val(users, iota)
    si, ip = lax.sort_key_val(items, iota)

    u, v = _make_gather_kernel(B, N, H)(su, up, si, ip, user_emb.T, item_emb.T)

    w1u = W1[:, :H].T
    w1v = W1[:, H:].T
    out = _make_mlp_call(B, H, 2048)(
        u, v, w1u, w1v, b1.reshape(1, -1), W2.T, b2.reshape(1, -1),
        W3.T, b3.reshape(1, -1), W4.T, b4.reshape(1, 1))
    return out


# speedup vs baseline: 3.0188x; 1.4389x over previous
import functools

import jax
import jax.numpy as jnp
from jax import lax
from jax.experimental import pallas as pl
from jax.experimental.pallas import tpu as pltpu
from jax.experimental.pallas import tpu_sc as plsc

NC = 2
NS = 16
NW = NC * NS
LANES = 128
CHUNK = 128


def _take16(x, idx):
    dn = lax.GatherDimensionNumbers(
        offset_dims=(), collapsed_slice_dims=(0,), start_index_map=(0,))
    return lax.gather(x, idx[:, None], dn, (1,),
                      mode=lax.GatherScatterMode.PROMISE_IN_BOUNDS)


def _make_gather_kernel(B, N, H):
    b_per_w = B // NW
    n_chunks = b_per_w // CHUNK
    mesh = plsc.VectorSubcoreMesh(core_axis_name="c", subcore_axis_name="s")

    @functools.partial(
        pl.kernel,
        mesh=mesh,
        compiler_params=pltpu.CompilerParams(needs_layout_passes=False),
        out_type=[
            jax.ShapeDtypeStruct((B, 2 * H), jnp.float32),
            jax.ShapeDtypeStruct((B, 2 * H), jnp.float32),
        ],
        scratch_types=[
            pltpu.VMEM((b_per_w,), jnp.int32),
            pltpu.VMEM((CHUNK,), jnp.int32),
            pltpu.VMEM((8, H, LANES), jnp.float32),
            pltpu.VMEM((CHUNK, 2 * H), jnp.float32),
            pltpu.SemaphoreType.DMA,
            pltpu.SemaphoreType.DMA,
        ],
    )
    def gather_k(su_hbm, up_hbm, si_hbm, ip_hbm, uembT_hbm, iembT_hbm,
                 uout, vout, sv_v, pos_v, ring, outbuf, fsem, ssem):
        wid = lax.axis_index("s") * NC + lax.axis_index("c")
        base = wid * b_per_w
        iota = lax.iota(jnp.int32, 16)
        prev = jnp.where(jnp.bitwise_and(iota, 7) == 0, iota, iota - 1)
        half_start = jnp.bitwise_and(iota, 7) == 0

        def one_table(sv_hbm, pos_hbm, tabT_hbm, out_hbm):
            pltpu.sync_copy(sv_hbm.at[pl.ds(base, b_per_w)], sv_v)
            for c in range(n_chunks):
                pltpu.sync_copy(pos_hbm.at[pl.ds(base + c * CHUNK, CHUNK)],
                                pos_v)

                def vec_body(v, _):
                    k0 = (c * CHUNK // 16 + v) * 16
                    idx_vec = sv_v[pl.ds(k0, 16)]
                    col_vec = lax.shift_right_logical(idx_vec, 7)
                    lane_vec = lax.bitwise_and(idx_vec, LANES - 1)
                    shifted = _take16(col_vec, prev)
                    is_new = jnp.logical_or(col_vec != shifted, half_start)
                    new_i = jnp.where(is_new, iota, 0)
                    slot_lane = plsc.cummax(new_i)
                    ring_slot = lax.bitwise_and(slot_lane, 7)

                    for half in range(2):
                        flags = []
                        for j in range(8):
                            L = half * 8 + j
                            sel = iota == L
                            newj = jnp.sum(jnp.where(sel, new_i, 0)) == L
                            colj = jnp.sum(jnp.where(sel, col_vec, 0))
                            flags.append(newj)

                            @pl.when(newj)
                            def _fetch():
                                off = pl.multiple_of(colj * LANES, LANES)
                                pltpu.async_copy(
                                    tabT_hbm.at[:, pl.ds(off, LANES)],
                                    ring.at[j], fsem)
                        for j in range(8):
                            @pl.when(flags[j])
                            def _drain():
                                pltpu.make_async_copy(
                                    tabT_hbm.at[:, pl.ds(0, LANES)],
                                    ring.at[j], fsem).wait()
                        for j in range(8):
                            L = half * 8 + j
                            Lv = jnp.full((16,), L, jnp.int32)
                            slot_s = _take16(ring_slot, Lv)
                            lane_s = _take16(lane_vec, Lv)
                            row = v * 16 + L
                            for jj in range(H // 16):
                                feats = iota + (16 * jj)
                                vals = plsc.load_gather(
                                    ring, [slot_s, feats, lane_s])
                                outbuf[row, pl.ds(16 * jj, 16)] = vals
                    return 0

                lax.fori_loop(0, CHUNK // 16, vec_body, 0)
                pltpu.async_copy(outbuf, out_hbm.at[pos_v], ssem).wait()

        one_table(su_hbm, up_hbm, uembT_hbm, uout)
        one_table(si_hbm, ip_hbm, iembT_hbm, vout)

    return gather_k


def _mlp_body(u_ref, v_ref, w1u_ref, w1v_ref, b1_ref, w2_ref, b2_ref,
              w3_ref, b3_ref, w4_ref, b4_ref, out_ref):
    f32 = jnp.float32
    H = 64
    x = jnp.dot(u_ref[:, :H], w1u_ref[...], preferred_element_type=f32)
    x += jnp.dot(v_ref[:, :H], w1v_ref[...], preferred_element_type=f32)
    h = jnp.maximum(x + b1_ref[...], 0.0)
    h = jnp.maximum(
        jnp.dot(h, w2_ref[...], preferred_element_type=f32) + b2_ref[...], 0.0)
    h = jnp.maximum(
        jnp.dot(h, w3_ref[...], preferred_element_type=f32) + b3_ref[...], 0.0)
    logit = jnp.dot(h, w4_ref[...], preferred_element_type=f32) + b4_ref[...]
    out_ref[...] = jax.nn.sigmoid(logit) * 9.0 + 1.0


def _make_mlp_call(B, H, BLK):
    grid = (B // BLK,)
    full = lambda shape: pl.BlockSpec(shape, lambda i: (0, 0))
    return pl.pallas_call(
        _mlp_body,
        grid=grid,
        in_specs=[
            pl.BlockSpec((BLK, 2 * H), lambda i: (i, 0)),
            pl.BlockSpec((BLK, 2 * H), lambda i: (i, 0)),
            full((H, 64)),
            full((H, 64)),
            full((1, 64)),
            full((64, 32)),
            full((1, 32)),
            full((32, 16)),
            full((1, 16)),
            full((16, 1)),
            full((1, 1)),
        ],
        out_specs=pl.BlockSpec((BLK, 1), lambda i: (i, 0)),
        out_shape=jax.ShapeDtypeStruct((B, 1), jnp.float32),
    )


def kernel(users, items, user_emb, item_emb, W1, b1, W2, b2, W3, b3, W4, b4):
    B = users.shape[0]
    N, H = user_emb.shape
    users = users.astype(jnp.int32)
    items = items.astype(jnp.int32)

    iota = lax.iota(jnp.int32, B)
    su, up = lax.sort_key_val(users, iota)
    si, ip = lax.sort_key_val(items, iota)

    u, v = _make_gather_kernel(B, N, H)(su, up, si, ip, user_emb.T, item_emb.T)

    w1u = W1[:, :H].T
    w1v = W1[:, H:].T
    out = _make_mlp_call(B, H, 2048)(
        u, v, w1u, w1v, b1.reshape(1, -1), W2.T, b2.reshape(1, -1),
        W3.T, b3.reshape(1, -1), W4.T, b4.reshape(1, 1))
    return out


# speedup vs baseline: 3.7120x; 1.2297x over previous
import functools

import jax
import jax.numpy as jnp
from jax import lax
from jax.experimental import pallas as pl
from jax.experimental.pallas import tpu as pltpu
from jax.experimental.pallas import tpu_sc as plsc

NC = 2
NS = 16
NW = NC * NS
LANES = 128
RING = 14
CHUNK = 64


def _take16(x, idx):
    dn = lax.GatherDimensionNumbers(
        offset_dims=(), collapsed_slice_dims=(0,), start_index_map=(0,))
    return lax.gather(x, idx[:, None], dn, (1,),
                      mode=lax.GatherScatterMode.PROMISE_IN_BOUNDS)


def _make_gather_kernel(B, N, H):
    b_per_w = B // NW
    n_q = b_per_w // 4
    n_body = (n_q - 8) // 12
    mesh = plsc.VectorSubcoreMesh(core_axis_name="c", subcore_axis_name="s")

    @functools.partial(
        pl.kernel,
        mesh=mesh,
        compiler_params=pltpu.CompilerParams(needs_layout_passes=False),
        out_type=[
            jax.ShapeDtypeStruct((B, 2 * H), jnp.float32),
            jax.ShapeDtypeStruct((B, 2 * H), jnp.float32),
        ],
        scratch_types=[
            pltpu.VMEM((b_per_w + 16,), jnp.int32),
            pltpu.VMEM((CHUNK,), jnp.int32),
            pltpu.VMEM((RING, H, LANES), jnp.float32),
            pltpu.VMEM((CHUNK, 2 * H), jnp.float32),
            pltpu.VMEM((3, 16), jnp.int32),
            pltpu.VMEM((3, 16), jnp.int32),
            pltpu.SemaphoreType.DMA,
            pltpu.SemaphoreType.DMA,
            pltpu.SemaphoreType.DMA,
            pltpu.SemaphoreType.DMA,
        ],
    )
    def gather_k(su_hbm, up_hbm, si_hbm, ip_hbm, uembT_hbm, iembT_hbm,
                 uout, vout, sv_v, pos_v, ring, outbuf, mslot, mlane,
                 f0, f1, f2, ssem):
        wid = lax.axis_index("s") * NC + lax.axis_index("c")
        base = wid * b_per_w
        iota = lax.iota(jnp.int32, 16)
        prev_perm = jnp.maximum(iota - 1, 0)
        fsems = [f0, f1, f2]

        def one_table(sv_hbm, pos_hbm, tabT_hbm, out_hbm):
            pltpu.sync_copy(sv_hbm.at[pl.ds(base, b_per_w)],
                            sv_v.at[pl.ds(0, b_per_w)])

            def step(i, t, carry, vec_cache, do_issue=True):
                prev_col, rank, n_m1, n_m2 = carry
                q = i * 12 + t
                v, qr = t // 4, t % 4

                t2 = (t - 2) % 12
                qr2 = t2 % 4
                row2 = t2 % 3
                sem2 = fsems[row2]
                for j in range(4):
                    @pl.when(n_m2 > j)
                    def _drain():
                        pltpu.make_async_copy(
                            tabT_hbm.at[:, pl.ds(0, LANES)],
                            ring.at[0], sem2).wait()

                @pl.when(q >= 2)
                def _extract():
                    ms = mslot[row2, :]
                    ml = mlane[row2, :]
                    for j in range(4):
                        L2 = jnp.full((16,), qr2 * 4 + j, jnp.int32)
                        slot_s = _take16(ms, L2)
                        lane_s = _take16(ml, L2)
                        r = lax.bitwise_and((q - 2) * 4 + j, CHUNK - 1)
                        for jj in range(H // 16):
                            feats = iota + (16 * jj)
                            vals = plsc.load_gather(
                                ring, [slot_s, feats, lane_s])
                            outbuf[r, pl.ds(16 * jj, 16)] = vals

                @pl.when(jnp.logical_and(
                    q >= 2, lax.bitwise_and(q - 2, 15) == 15))
                def _scatter():
                    chunk = lax.shift_right_logical(q - 2, 4)
                    pltpu.sync_copy(
                        pos_hbm.at[pl.ds(base + chunk * CHUNK, CHUNK)],
                        pos_v)
                    pltpu.async_copy(outbuf, out_hbm.at[pos_v], ssem).wait()

                if not do_issue:
                    return (prev_col, rank, jnp.int32(0), n_m1), vec_cache

                if qr == 0:
                    idx_vec = sv_v[pl.ds((i * 3 + v) * 16, 16)]
                    col_vec = lax.shift_right_logical(idx_vec, 7)
                    lane_vec = lax.bitwise_and(idx_vec, LANES - 1)
                    shifted = _take16(col_vec, prev_perm)
                    prev_splat = jnp.full((16,), 1, jnp.int32) * prev_col
                    cmp = jnp.where(iota == 0, prev_splat, shifted)
                    is_new_vec = (col_vec != cmp).astype(jnp.int32)
                    vec_cache = (col_vec, lane_vec, is_new_vec)
                col_vec, lane_vec, is_new_vec = vec_cache

                qmask = jnp.logical_and(iota >= qr * 4, iota < qr * 4 + 4)
                new_m = jnp.where(qmask, is_new_vec, 0)
                nq = jnp.sum(new_m)
                cum = plsc.cumsum(new_m)
                rank_splat = jnp.full((16,), 1, jnp.int32) * rank
                slot_vec = lax.rem(rank_splat + cum + (RING - 1), RING)
                row_t = t % 3
                mslot[row_t, :] = slot_vec
                mlane[row_t, :] = lane_vec
                semq = fsems[row_t]
                for j in range(4):
                    L = qr * 4 + j
                    sel = iota == L
                    newj = jnp.sum(jnp.where(sel, new_m, 0)) > 0
                    colj = jnp.sum(jnp.where(sel, col_vec, 0))
                    slotj = jnp.sum(jnp.where(sel, slot_vec, 0))

                    @pl.when(newj)
                    def _fetch():
                        off = pl.multiple_of(colj * LANES, LANES)
                        pltpu.async_copy(
                            tabT_hbm.at[:, pl.ds(off, LANES)],
                            ring.at[slotj], semq)

                if qr == 3:
                    prev_col = jnp.sum(jnp.where(iota == 15, col_vec, 0))
                rank = lax.rem(rank + nq, RING)
                return (prev_col, rank, nq, n_m1), vec_cache

            def body(i, carry):
                vc = None
                for t in range(12):
                    carry, vc = step(i, t, carry, vc)
                return carry

            carry = (jnp.int32(-1), jnp.int32(0), jnp.int32(0), jnp.int32(0))
            carry = lax.fori_loop(0, n_body, body, carry)
            i_tail = jnp.int32(n_body)
            vc = None
            for t in range(n_q - n_body * 12):
                carry, vc = step(i_tail, t, carry, vc)
            for t in range(n_q - n_body * 12, n_q - n_body * 12 + 2):
                carry, vc = step(i_tail, t, carry, vc, do_issue=False)

        one_table(su_hbm, up_hbm, uembT_hbm, uout)
        one_table(si_hbm, ip_hbm, iembT_hbm, vout)

    return gather_k


def _mlp_body(u_ref, v_ref, w1u_ref, w1v_ref, b1_ref, w2_ref, b2_ref,
              w3_ref, b3_ref, w4_ref, b4_ref, out_ref):
    f32 = jnp.float32
    H = 64
    x = jnp.dot(u_ref[:, :H], w1u_ref[...], preferred_element_type=f32)
    x += jnp.dot(v_ref[:, :H], w1v_ref[...], preferred_element_type=f32)
    h = jnp.maximum(x + b1_ref[...], 0.0)
    h = jnp.maximum(
        jnp.dot(h, w2_ref[...], preferred_element_type=f32) + b2_ref[...], 0.0)
    h = jnp.maximum(
        jnp.dot(h, w3_ref[...], preferred_element_type=f32) + b3_ref[...], 0.0)
    logit = jnp.dot(h, w4_ref[...], preferred_element_type=f32) + b4_ref[...]
    out_ref[...] = jax.nn.sigmoid(logit) * 9.0 + 1.0


def _make_mlp_call(B, H, BLK):
    grid = (B // BLK,)
    full = lambda shape: pl.BlockSpec(shape, lambda i: (0, 0))
    return pl.pallas_call(
        _mlp_body,
        grid=grid,
        in_specs=[
            pl.BlockSpec((BLK, 2 * H), lambda i: (i, 0)),
            pl.BlockSpec((BLK, 2 * H), lambda i: (i, 0)),
            full((H, 64)),
            full((H, 64)),
            full((1, 64)),
            full((64, 32)),
            full((1, 32)),
            full((32, 16)),
            full((1, 16)),
            full((16, 1)),
            full((1, 1)),
        ],
        out_specs=pl.BlockSpec((BLK, 1), lambda i: (i, 0)),
        out_shape=jax.ShapeDtypeStruct((B, 1), jnp.float32),
    )


def kernel(users, items, user_emb, item_emb, W1, b1, W2, b2, W3, b3, W4, b4):
    B = users.shape[0]
    N, H = user_emb.shape
    users = users.astype(jnp.int32)
    items = items.astype(jnp.int32)

    iota = lax.iota(jnp.int32, B)
    su, up = lax.sort_key_val(users, iota)
    si, ip = lax.sort_key_val(items, iota)

    u, v = _make_gather_kernel(B, N, H)(su, up, si, ip, user_emb.T, item_emb.T)

    w1u = W1[:, :H].T
    w1v = W1[:, H:].T
    out = _make_mlp_call(B, H, 2048)(
        u, v, w1u, w1v, b1.reshape(1, -1), W2.T, b2.reshape(1, -1),
        W3.T, b3.reshape(1, -1), W4.T, b4.reshape(1, 1))
    return out


# speedup vs baseline: 4.2665x; 1.1494x over previous
import functools

import jax
import jax.numpy as jnp
from jax import lax
from jax.experimental import pallas as pl
from jax.experimental.pallas import tpu as pltpu
from jax.experimental.pallas import tpu_sc as plsc

NC = 2
NS = 16
NW = NC * NS
LANES = 128
RING = 14
CHUNK = 64


def _take16(x, idx):
    dn = lax.GatherDimensionNumbers(
        offset_dims=(), collapsed_slice_dims=(0,), start_index_map=(0,))
    return lax.gather(x, idx[:, None], dn, (1,),
                      mode=lax.GatherScatterMode.PROMISE_IN_BOUNDS)


def _make_gather_kernel(B, N, H):
    b_per_w = B // NW
    n_q = b_per_w // 4
    n_body = (n_q - 8) // 12
    mesh = plsc.VectorSubcoreMesh(core_axis_name="c", subcore_axis_name="s")

    @functools.partial(
        pl.kernel,
        mesh=mesh,
        compiler_params=pltpu.CompilerParams(needs_layout_passes=False),
        out_type=[
            jax.ShapeDtypeStruct((B, 2 * H), jnp.float32),
            jax.ShapeDtypeStruct((B, 2 * H), jnp.float32),
        ],
        scratch_types=[
            pltpu.VMEM((b_per_w + 16,), jnp.int32),
            pltpu.VMEM((CHUNK,), jnp.int32),
            pltpu.VMEM((RING, H, LANES), jnp.float32),
            pltpu.VMEM((CHUNK, 2 * H), jnp.float32),
            pltpu.VMEM((4, 16), jnp.int32),
            pltpu.VMEM((4, 16), jnp.int32),
            pltpu.SemaphoreType.DMA,
            pltpu.SemaphoreType.DMA,
            pltpu.SemaphoreType.DMA,
            pltpu.SemaphoreType.DMA,
            pltpu.SemaphoreType.DMA,
        ],
    )
    def gather_k(su_hbm, up_hbm, si_hbm, ip_hbm, uembT_hbm, iembT_hbm,
                 uout, vout, sv_v, pos_v, ring, outbuf, mslot, mlane,
                 f0, f1, f2, f3, ssem):
        wid = lax.axis_index("s") * NC + lax.axis_index("c")
        base = wid * b_per_w
        iota = lax.iota(jnp.int32, 16)
        prev_perm = jnp.maximum(iota - 1, 0)
        fsems = [f0, f1, f2, f3]

        def one_table(sv_hbm, pos_hbm, tabT_hbm, out_hbm):
            pltpu.sync_copy(sv_hbm.at[pl.ds(base, b_per_w)],
                            sv_v.at[pl.ds(0, b_per_w)])

            def step(i, t, carry, vec_cache, do_issue=True):
                prev_col, rank, n_m1, n_m2, n_m3 = carry
                q = i * 12 + t
                v, qr = t // 4, t % 4

                t2 = (t - 3) % 12
                qr2 = t2 % 4
                row2 = t2 % 4
                sem2 = fsems[row2]
                for j in range(4):
                    @pl.when(n_m3 > j)
                    def _drain():
                        pltpu.make_async_copy(
                            tabT_hbm.at[:, pl.ds(0, LANES)],
                            ring.at[0], sem2).wait()

                @pl.when(q >= 3)
                def _extract():
                    ms = mslot[row2, :]
                    ml = mlane[row2, :]
                    for j in range(4):
                        L2 = jnp.full((16,), qr2 * 4 + j, jnp.int32)
                        slot_s = _take16(ms, L2)
                        lane_s = _take16(ml, L2)
                        r = lax.bitwise_and((q - 3) * 4 + j, CHUNK - 1)
                        for jj in range(H // 16):
                            feats = iota + (16 * jj)
                            vals = plsc.load_gather(
                                ring, [slot_s, feats, lane_s])
                            outbuf[r, pl.ds(16 * jj, 16)] = vals

                @pl.when(jnp.logical_and(
                    q >= 3, lax.bitwise_and(q - 3, 15) == 15))
                def _scatter():
                    chunk = lax.shift_right_logical(q - 3, 4)
                    pltpu.sync_copy(
                        pos_hbm.at[pl.ds(base + chunk * CHUNK, CHUNK)],
                        pos_v)
                    pltpu.async_copy(outbuf, out_hbm.at[pos_v], ssem).wait()

                if not do_issue:
                    return (prev_col, rank, jnp.int32(0), n_m1, n_m2), \
                        vec_cache

                if qr == 0:
                    idx_vec = sv_v[pl.ds((i * 3 + v) * 16, 16)]
                    col_vec = lax.shift_right_logical(idx_vec, 7)
                    lane_vec = lax.bitwise_and(idx_vec, LANES - 1)
                    shifted = _take16(col_vec, prev_perm)
                    prev_splat = jnp.full((16,), 1, jnp.int32) * prev_col
                    cmp = jnp.where(iota == 0, prev_splat, shifted)
                    is_new_vec = (col_vec != cmp).astype(jnp.int32)
                    vec_cache = (col_vec, lane_vec, is_new_vec)
                col_vec, lane_vec, is_new_vec = vec_cache

                qmask = jnp.logical_and(iota >= qr * 4, iota < qr * 4 + 4)
                new_m = jnp.where(qmask, is_new_vec, 0)
                nq = jnp.sum(new_m)
                cum = plsc.cumsum(new_m)
                rank_splat = jnp.full((16,), 1, jnp.int32) * rank
                slot_vec = lax.rem(rank_splat + cum + (RING - 1), RING)
                row_t = t % 4
                mslot[row_t, :] = slot_vec
                mlane[row_t, :] = lane_vec
                semq = fsems[row_t]
                for j in range(4):
                    L = qr * 4 + j
                    sel = iota == L
                    newj = jnp.sum(jnp.where(sel, new_m, 0)) > 0
                    colj = jnp.sum(jnp.where(sel, col_vec, 0))
                    slotj = jnp.sum(jnp.where(sel, slot_vec, 0))

                    @pl.when(newj)
                    def _fetch():
                        off = pl.multiple_of(colj * LANES, LANES)
                        pltpu.async_copy(
                            tabT_hbm.at[:, pl.ds(off, LANES)],
                            ring.at[slotj], semq)

                if qr == 3:
                    prev_col = jnp.sum(jnp.where(iota == 15, col_vec, 0))
                rank = lax.rem(rank + nq, RING)
                return (prev_col, rank, nq, n_m1, n_m2), vec_cache

            def body(i, carry):
                vc = None
                for t in range(12):
                    carry, vc = step(i, t, carry, vc)
                return carry

            carry = (jnp.int32(-1), jnp.int32(0), jnp.int32(0),
                     jnp.int32(0), jnp.int32(0))
            carry = lax.fori_loop(0, n_body, body, carry)
            i_tail = jnp.int32(n_body)
            vc = None
            for t in range(n_q - n_body * 12):
                carry, vc = step(i_tail, t, carry, vc)
            for t in range(n_q - n_body * 12, n_q - n_body * 12 + 3):
                carry, vc = step(i_tail, t, carry, vc, do_issue=False)

        one_table(su_hbm, up_hbm, uembT_hbm, uout)
        one_table(si_hbm, ip_hbm, iembT_hbm, vout)

    return gather_k


def _mlp_body(u_ref, v_ref, w1u_ref, w1v_ref, b1_ref, w2_ref, b2_ref,
              w3_ref, b3_ref, w4_ref, b4_ref, out_ref):
    f32 = jnp.float32
    H = 64
    x = jnp.dot(u_ref[:, :H], w1u_ref[...], preferred_element_type=f32)
    x += jnp.dot(v_ref[:, :H], w1v_ref[...], preferred_element_type=f32)
    h = jnp.maximum(x + b1_ref[...], 0.0)
    h = jnp.maximum(
        jnp.dot(h, w2_ref[...], preferred_element_type=f32) + b2_ref[...], 0.0)
    h = jnp.maximum(
        jnp.dot(h, w3_ref[...], preferred_element_type=f32) + b3_ref[...], 0.0)
    logit = jnp.dot(h, w4_ref[...], preferred_element_type=f32) + b4_ref[...]
    out_ref[...] = jax.nn.sigmoid(logit) * 9.0 + 1.0


def _make_mlp_call(B, H, BLK):
    grid = (B // BLK,)
    full = lambda shape: pl.BlockSpec(shape, lambda i: (0, 0))
    return pl.pallas_call(
        _mlp_body,
        grid=grid,
        in_specs=[
            pl.BlockSpec((BLK, 2 * H), lambda i: (i, 0)),
            pl.BlockSpec((BLK, 2 * H), lambda i: (i, 0)),
            full((H, 64)),
            full((H, 64)),
            full((1, 64)),
            full((64, 32)),
            full((1, 32)),
            full((32, 16)),
            full((1, 16)),
            full((16, 1)),
            full((1, 1)),
        ],
        out_specs=pl.BlockSpec((BLK, 1), lambda i: (i, 0)),
        out_shape=jax.ShapeDtypeStruct((B, 1), jnp.float32),
    )


def kernel(users, items, user_emb, item_emb, W1, b1, W2, b2, W3, b3, W4, b4):
    B = users.shape[0]
    N, H = user_emb.shape
    users = users.astype(jnp.int32)
    items = items.astype(jnp.int32)

    iota = lax.iota(jnp.int32, B)
    su, up = lax.sort_key_val(users, iota)
    si, ip = lax.sort_key_val(items, iota)

    u, v = _make_gather_kernel(B, N, H)(su, up, si, ip, user_emb.T, item_emb.T)

    w1u = W1[:, :H].T
    w1v = W1[:, H:].T
    out = _make_mlp_call(B, H, 2048)(
        u, v, w1u, w1v, b1.reshape(1, -1), W2.T, b2.reshape(1, -1),
        W3.T, b3.reshape(1, -1), W4.T, b4.reshape(1, 1))
    return out


# speedup vs baseline: 4.4199x; 1.0359x over previous
import functools

import jax
import jax.numpy as jnp
from jax import lax
from jax.experimental import pallas as pl
from jax.experimental.pallas import tpu as pltpu
from jax.experimental.pallas import tpu_sc as plsc

NC = 2
NS = 16
NW = NC * NS
LANES = 128
RING = 13
CHUNK = 64


def _take16(x, idx):
    dn = lax.GatherDimensionNumbers(
        offset_dims=(), collapsed_slice_dims=(0,), start_index_map=(0,))
    return lax.gather(x, idx[:, None], dn, (1,),
                      mode=lax.GatherScatterMode.PROMISE_IN_BOUNDS)


def _make_gather_kernel(B, N, H):
    b_per_w = B // NW
    n_q = b_per_w // 4
    mesh = plsc.VectorSubcoreMesh(core_axis_name="c", subcore_axis_name="s")

    @functools.partial(
        pl.kernel,
        mesh=mesh,
        compiler_params=pltpu.CompilerParams(needs_layout_passes=False),
        out_type=[
            jax.ShapeDtypeStruct((B, 2 * H), jnp.float32),
            jax.ShapeDtypeStruct((B, 2 * H), jnp.float32),
        ],
        scratch_types=[
            pltpu.VMEM((b_per_w + 16,), jnp.int32),
            pltpu.VMEM((CHUNK,), jnp.int32),
            pltpu.VMEM((RING, H, LANES), jnp.float32),
            pltpu.VMEM((2 * CHUNK, 2 * H), jnp.float32),
            pltpu.VMEM((4, 16), jnp.int32),
            pltpu.VMEM((4, 16), jnp.int32),
            pltpu.SemaphoreType.DMA,
            pltpu.SemaphoreType.DMA,
            pltpu.SemaphoreType.DMA,
            pltpu.SemaphoreType.DMA,
            pltpu.SemaphoreType.DMA,
        ],
    )
    def gather_k(su_hbm, up_hbm, si_hbm, ip_hbm, uembT_hbm, iembT_hbm,
                 uout, vout, sv_v, pos_v, ring, outbuf, mslot, mlane,
                 f0, f1, f2, f3, ssem):
        wid = lax.axis_index("s") * NC + lax.axis_index("c")
        base = wid * b_per_w
        iota = lax.iota(jnp.int32, 16)
        prev_perm = jnp.maximum(iota - 1, 0)
        fsems = [f0, f1, f2, f3]

        def one_table(sv_hbm, pos_hbm, tabT_hbm, out_hbm):
            pltpu.sync_copy(sv_hbm.at[pl.ds(base, b_per_w)],
                            sv_v.at[pl.ds(0, b_per_w)])

            def step(i, t, carry, vec_cache, do_issue=True):
                prev_col, rank, n_m1, n_m2, n_m3 = carry
                q = i * 4 + t
                qr = t % 4

                qr2 = (t - 3) % 4
                row2 = qr2
                sem2 = fsems[row2]
                for j in range(4):
                    @pl.when(n_m3 > j)
                    def _drain():
                        pltpu.make_async_copy(
                            tabT_hbm.at[:, pl.ds(0, LANES)],
                            ring.at[0], sem2).wait()

                @pl.when(q >= 3)
                def _extract():
                    ms = mslot[row2, :]
                    ml = mlane[row2, :]
                    for j in range(4):
                        L2 = jnp.full((16,), qr2 * 4 + j, jnp.int32)
                        slot_s = _take16(ms, L2)
                        lane_s = _take16(ml, L2)
                        r = lax.bitwise_and((q - 3) * 4 + j,
                                            2 * CHUNK - 1)
                        for jj in range(H // 16):
                            feats = iota + (16 * jj)
                            vals = plsc.load_gather(
                                ring, [slot_s, feats, lane_s])
                            outbuf[r, pl.ds(16 * jj, 16)] = vals

                @pl.when(jnp.logical_and(
                    q >= 3, lax.bitwise_and(q - 3, 15) == 15))
                def _scatter():
                    chunk = lax.shift_right_logical(q - 3, 4)

                    @pl.when(chunk >= 1)
                    def _drain_prev():
                        pltpu.make_async_copy(
                            tabT_hbm.at[:, pl.ds(0, LANES)],
                            outbuf.at[pl.ds(0, CHUNK)], ssem).wait()

                    pltpu.sync_copy(
                        pos_hbm.at[pl.ds(base + chunk * CHUNK, CHUNK)],
                        pos_v)
                    off = pl.multiple_of(
                        lax.bitwise_and(chunk, 1) * CHUNK, CHUNK)
                    pltpu.async_copy(outbuf.at[pl.ds(off, CHUNK)],
                                     out_hbm.at[pos_v], ssem)

                if not do_issue:
                    return (prev_col, rank, jnp.int32(0), n_m1, n_m2), \
                        vec_cache

                if qr == 0:
                    idx_vec = sv_v[pl.ds(i * 16, 16)]
                    col_vec = lax.shift_right_logical(idx_vec, 7)
                    lane_vec = lax.bitwise_and(idx_vec, LANES - 1)
                    shifted = _take16(col_vec, prev_perm)
                    prev_splat = jnp.full((16,), 1, jnp.int32) * prev_col
                    cmp = jnp.where(iota == 0, prev_splat, shifted)
                    is_new_vec = (col_vec != cmp).astype(jnp.int32)
                    vec_cache = (col_vec, lane_vec, is_new_vec)
                col_vec, lane_vec, is_new_vec = vec_cache

                qmask = jnp.logical_and(iota >= qr * 4, iota < qr * 4 + 4)
                new_m = jnp.where(qmask, is_new_vec, 0)
                nq = jnp.sum(new_m)
                cum = plsc.cumsum(new_m)
                rank_splat = jnp.full((16,), 1, jnp.int32) * rank
                slot_vec = lax.rem(rank_splat + cum + (RING - 1), RING)
                row_t = t % 4
                mslot[row_t, :] = slot_vec
                mlane[row_t, :] = lane_vec
                semq = fsems[row_t]
                for j in range(4):
                    L = qr * 4 + j
                    sel = iota == L
                    newj = jnp.sum(jnp.where(sel, new_m, 0)) > 0
                    colj = jnp.sum(jnp.where(sel, col_vec, 0))
                    slotj = jnp.sum(jnp.where(sel, slot_vec, 0))

                    @pl.when(newj)
                    def _fetch():
                        off = pl.multiple_of(colj * LANES, LANES)
                        pltpu.async_copy(
                            tabT_hbm.at[:, pl.ds(off, LANES)],
                            ring.at[slotj], semq)

                if qr == 3:
                    prev_col = jnp.sum(jnp.where(iota == 15, col_vec, 0))
                rank = lax.rem(rank + nq, RING)
                return (prev_col, rank, nq, n_m1, n_m2), vec_cache

            def body(i, carry):
                vc = None
                for t in range(4):
                    carry, vc = step(i, t, carry, vc)
                return carry

            carry = (jnp.int32(-1), jnp.int32(0), jnp.int32(0),
                     jnp.int32(0), jnp.int32(0))
            carry = lax.fori_loop(0, n_q // 4, body, carry)
            i_tail = jnp.int32(n_q // 4)
            vc = None
            for t in range(3):
                carry, vc = step(i_tail, t, carry, vc, do_issue=False)
            pltpu.make_async_copy(
                uembT_hbm.at[:, pl.ds(0, LANES)],
                outbuf.at[pl.ds(0, CHUNK)], ssem).wait()

        one_table(su_hbm, up_hbm, uembT_hbm, uout)
        one_table(si_hbm, ip_hbm, iembT_hbm, vout)

    return gather_k


def _mlp_body(u_ref, v_ref, w1u_ref, w1v_ref, b1_ref, w2_ref, b2_ref,
              w3_ref, b3_ref, w4_ref, b4_ref, out_ref):
    f32 = jnp.float32
    H = 64
    x = jnp.dot(u_ref[:, :H], w1u_ref[...], preferred_element_type=f32)
    x += jnp.dot(v_ref[:, :H], w1v_ref[...], preferred_element_type=f32)
    h = jnp.maximum(x + b1_ref[...], 0.0)
    h = jnp.maximum(
        jnp.dot(h, w2_ref[...], preferred_element_type=f32) + b2_ref[...], 0.0)
    h = jnp.maximum(
        jnp.dot(h, w3_ref[...], preferred_element_type=f32) + b3_ref[...], 0.0)
    logit = jnp.dot(h, w4_ref[...], preferred_element_type=f32) + b4_ref[...]
    out_ref[...] = jax.nn.sigmoid(logit) * 9.0 + 1.0


def _make_mlp_call(B, H, BLK):
    grid = (B // BLK,)
    full = lambda shape: pl.BlockSpec(shape, lambda i: (0, 0))
    return pl.pallas_call(
        _mlp_body,
        grid=grid,
        in_specs=[
            pl.BlockSpec((BLK, 2 * H), lambda i: (i, 0)),
            pl.BlockSpec((BLK, 2 * H), lambda i: (i, 0)),
            full((H, 64)),
            full((H, 64)),
            full((1, 64)),
            full((64, 32)),
            full((1, 32)),
            full((32, 16)),
            full((1, 16)),
            full((16, 1)),
            full((1, 1)),
        ],
        out_specs=pl.BlockSpec((BLK, 1), lambda i: (i, 0)),
        out_shape=jax.ShapeDtypeStruct((B, 1), jnp.float32),
    )


def kernel(users, items, user_emb, item_emb, W1, b1, W2, b2, W3, b3, W4, b4):
    B = users.shape[0]
    N, H = user_emb.shape
    users = users.astype(jnp.int32)
    items = items.astype(jnp.int32)

    iota = lax.iota(jnp.int32, B)
    su, up = lax.sort_key_val(users, iota)
    si, ip = lax.sort_key_val(items, iota)

    u, v = _make_gather_kernel(B, N, H)(su, up, si, ip, user_emb.T, item_emb.T)

    w1u = W1[:, :H].T
    w1v = W1[:, H:].T
    out = _make_mlp_call(B, H, 2048)(
        u, v, w1u, w1v, b1.reshape(1, -1), W2.T, b2.reshape(1, -1),
        W3.T, b3.reshape(1, -1), W4.T, b4.reshape(1, 1))
    return out


# speedup vs baseline: 4.4554x; 1.0080x over previous
import functools

import jax
import jax.numpy as jnp
from jax import lax
from jax.experimental import pallas as pl
from jax.experimental.pallas import tpu as pltpu
from jax.experimental.pallas import tpu_sc as plsc

NC = 2
NS = 16
NW = NC * NS
LANES = 128
RING = 13
CHUNK = 64


def _take16(x, idx):
    dn = lax.GatherDimensionNumbers(
        offset_dims=(), collapsed_slice_dims=(0,), start_index_map=(0,))
    return lax.gather(x, idx[:, None], dn, (1,),
                      mode=lax.GatherScatterMode.PROMISE_IN_BOUNDS)


def _make_gather_kernel(B, N, H):
    b_per_w = B // NW
    n_q = b_per_w // 4
    mesh = plsc.VectorSubcoreMesh(core_axis_name="c", subcore_axis_name="s")

    @functools.partial(
        pl.kernel,
        mesh=mesh,
        compiler_params=pltpu.CompilerParams(needs_layout_passes=False),
        out_type=[
            jax.ShapeDtypeStruct((B, 2 * H), jnp.float32),
            jax.ShapeDtypeStruct((B, 2 * H), jnp.float32),
        ],
        scratch_types=[
            pltpu.VMEM((b_per_w + 16,), jnp.int32),
            pltpu.VMEM((CHUNK,), jnp.int32),
            pltpu.VMEM((RING, H, LANES), jnp.float32),
            pltpu.VMEM((2 * CHUNK, 2 * H), jnp.float32),
            pltpu.VMEM((4, 16), jnp.int32),
            pltpu.VMEM((4, 16), jnp.int32),
            pltpu.SemaphoreType.DMA,
            pltpu.SemaphoreType.DMA,
            pltpu.SemaphoreType.DMA,
            pltpu.SemaphoreType.DMA,
            pltpu.SemaphoreType.DMA,
        ],
    )
    def gather_k(su_hbm, up_hbm, si_hbm, ip_hbm, uembT_hbm, iembT_hbm,
                 uout, vout, sv_v, pos_v, ring, outbuf, mslot, mlane,
                 f0, f1, f2, f3, ssem):
        wid = lax.axis_index("s") * NC + lax.axis_index("c")
        base = wid * b_per_w
        iota = lax.iota(jnp.int32, 16)
        prev_perm = jnp.maximum(iota - 1, 0)
        fsems = [f0, f1, f2, f3]

        def one_table(sv_hbm, pos_hbm, tabT_hbm, out_hbm):
            pltpu.sync_copy(sv_hbm.at[pl.ds(base, b_per_w)],
                            sv_v.at[pl.ds(0, b_per_w)])

            def step(i, t, carry, vec_cache, do_issue=True):
                prev_col, rank, n_m1, n_m2, n_m3 = carry
                q = i * 4 + t
                qr = t % 4

                qr2 = (t - 3) % 4
                row2 = qr2
                sem2 = fsems[row2]
                for j in range(4):
                    @pl.when(n_m3 > j)
                    def _drain():
                        pltpu.make_async_copy(
                            tabT_hbm.at[:, pl.ds(0, LANES)],
                            ring.at[0], sem2).wait()

                @pl.when(q >= 3)
                def _extract():
                    ms = mslot[row2, :]
                    ml = mlane[row2, :]
                    for j in range(4):
                        L2 = jnp.full((16,), qr2 * 4 + j, jnp.int32)
                        slot_s = _take16(ms, L2)
                        lane_s = _take16(ml, L2)
                        r = lax.bitwise_and((q - 3) * 4 + j,
                                            2 * CHUNK - 1)
                        for jj in range(H // 16):
                            feats = iota + (16 * jj)
                            vals = plsc.load_gather(
                                ring, [slot_s, feats, lane_s])
                            outbuf[r, pl.ds(16 * jj, 16)] = vals

                @pl.when(jnp.logical_and(
                    q >= 3, lax.bitwise_and(q - 3, 15) == 15))
                def _scatter():
                    chunk = lax.shift_right_logical(q - 3, 4)

                    @pl.when(chunk >= 1)
                    def _drain_prev():
                        pltpu.make_async_copy(
                            tabT_hbm.at[:, pl.ds(0, LANES)],
                            outbuf.at[pl.ds(0, CHUNK)], ssem).wait()

                    pltpu.sync_copy(
                        pos_hbm.at[pl.ds(base + chunk * CHUNK, CHUNK)],
                        pos_v)
                    off = pl.multiple_of(
                        lax.bitwise_and(chunk, 1) * CHUNK, CHUNK)
                    pltpu.async_copy(outbuf.at[pl.ds(off, CHUNK)],
                                     out_hbm.at[pos_v], ssem)

                if not do_issue:
                    return (prev_col, rank, jnp.int32(0), n_m1, n_m2), \
                        vec_cache

                if qr == 0:
                    idx_vec = sv_v[pl.ds(i * 16, 16)]
                    col_vec = lax.shift_right_logical(idx_vec, 7)
                    lane_vec = lax.bitwise_and(idx_vec, LANES - 1)
                    shifted = _take16(col_vec, prev_perm)
                    prev_splat = jnp.full((16,), 1, jnp.int32) * prev_col
                    cmp = jnp.where(iota == 0, prev_splat, shifted)
                    is_new_vec = (col_vec != cmp).astype(jnp.int32)
                    vec_cache = (col_vec, lane_vec, is_new_vec)
                col_vec, lane_vec, is_new_vec = vec_cache

                qmask = jnp.logical_and(iota >= qr * 4, iota < qr * 4 + 4)
                new_m = jnp.where(qmask, is_new_vec, 0)
                nq = jnp.sum(new_m)
                cum = plsc.cumsum(new_m)
                rank_splat = jnp.full((16,), 1, jnp.int32) * rank
                slot_vec = lax.rem(rank_splat + cum + (RING - 1), RING)
                row_t = t % 4
                mslot[row_t, :] = slot_vec
                mlane[row_t, :] = lane_vec
                semq = fsems[row_t]
                for j in range(4):
                    L = qr * 4 + j
                    sel = iota == L
                    newj = jnp.sum(jnp.where(sel, new_m, 0)) > 0
                    colj = jnp.sum(jnp.where(sel, col_vec, 0))
                    slotj = jnp.sum(jnp.where(sel, slot_vec, 0))

                    @pl.when(newj)
                    def _fetch():
                        off = pl.multiple_of(colj * LANES, LANES)
                        pltpu.async_copy(
                            tabT_hbm.at[:, pl.ds(off, LANES)],
                            ring.at[slotj], semq)

                if qr == 3:
                    prev_col = jnp.sum(jnp.where(iota == 15, col_vec, 0))
                rank = lax.rem(rank + nq, RING)
                return (prev_col, rank, nq, n_m1, n_m2), vec_cache

            def body(i, carry):
                vc = None
                for t in range(4):
                    carry, vc = step(i, t, carry, vc)
                return carry

            carry = (jnp.int32(-1), jnp.int32(0), jnp.int32(0),
                     jnp.int32(0), jnp.int32(0))
            carry = lax.fori_loop(0, n_q // 4, body, carry)
            i_tail = jnp.int32(n_q // 4)
            vc = None
            for t in range(3):
                carry, vc = step(i_tail, t, carry, vc, do_issue=False)
            pltpu.make_async_copy(
                uembT_hbm.at[:, pl.ds(0, LANES)],
                outbuf.at[pl.ds(0, CHUNK)], ssem).wait()

        one_table(su_hbm, up_hbm, uembT_hbm, uout)
        one_table(si_hbm, ip_hbm, iembT_hbm, vout)

    return gather_k


def _mlp_body(u_ref, v_ref, w1u_ref, w1v_ref, b1_ref, w2_ref, b2_ref,
              w3_ref, b3_ref, w4_ref, b4_ref, out_ref):
    f32 = jnp.float32
    H = 64
    x = jnp.dot(u_ref[:, :H], w1u_ref[...], preferred_element_type=f32)
    x += jnp.dot(v_ref[:, :H], w1v_ref[...], preferred_element_type=f32)
    h = jnp.maximum(x + b1_ref[...], 0.0)
    h = jnp.maximum(
        jnp.dot(h, w2_ref[...], preferred_element_type=f32) + b2_ref[...], 0.0)
    h = jnp.maximum(
        jnp.dot(h, w3_ref[...], preferred_element_type=f32) + b3_ref[...], 0.0)
    logit = jnp.dot(h, w4_ref[...], preferred_element_type=f32) + b4_ref[...]
    out_ref[...] = jax.nn.sigmoid(logit) * 9.0 + 1.0


def _make_mlp_call(B, H, BLK):
    grid = (B // BLK,)
    full = lambda shape: pl.BlockSpec(shape, lambda i: (0, 0))
    return pl.pallas_call(
        _mlp_body,
        grid=grid,
        in_specs=[
            pl.BlockSpec((BLK, 2 * H), lambda i: (i, 0)),
            pl.BlockSpec((BLK, 2 * H), lambda i: (i, 0)),
            full((H, 64)),
            full((H, 64)),
            full((1, 64)),
            full((64, 32)),
            full((1, 32)),
            full((32, 16)),
            full((1, 16)),
            full((16, 1)),
            full((1, 1)),
        ],
        out_specs=pl.BlockSpec((BLK, 1), lambda i: (i, 0)),
        out_shape=jax.ShapeDtypeStruct((B, 1), jnp.float32),
    )


def kernel(users, items, user_emb, item_emb, W1, b1, W2, b2, W3, b3, W4, b4):
    B = users.shape[0]
    N, H = user_emb.shape
    users = users.astype(jnp.int32)
    items = items.astype(jnp.int32)

    iota = lax.iota(jnp.int32, B)
    su, up = lax.sort_key_val(users, iota)
    si, ip = lax.sort_key_val(items, iota)

    u, v = _make_gather_kernel(B, N, H)(su, up, si, ip, user_emb.T, item_emb.T)

    w1u = W1[:, :H].T
    w1v = W1[:, H:].T
    out = _make_mlp_call(B, H, 8192)(
        u, v, w1u, w1v, b1.reshape(1, -1), W2.T, b2.reshape(1, -1),
        W3.T, b3.reshape(1, -1), W4.T, b4.reshape(1, 1))
    return out
